# initial kernel scaffold (unmeasured)
import jax
import jax.numpy as jnp
from jax import lax
from jax.experimental import pallas as pl
from jax.experimental.pallas import tpu as pltpu

N_DEV = 4


def _ring_allreduce(x):
    t, d = x.shape
    chunk = t // N_DEV
    n_hops = 2 * (N_DEV - 1)

    def body(x_ref, out_ref, comm_ref, send_sems, recv_sems):
        my = lax.axis_index("i")
        left = (my + N_DEV - 1) % N_DEV
        right = (my + 1) % N_DEV

        barrier_sem = pltpu.get_barrier_semaphore()
        for nbr in (left, right):
            pl.semaphore_signal(
                barrier_sem, inc=1,
                device_id=(nbr,), device_id_type=pl.DeviceIdType.MESH,
            )
        pl.semaphore_wait(barrier_sem, 2)

        out_ref[...] = x_ref[...]

        for s in range(N_DEV - 1):
            send_idx = (my + N_DEV - s) % N_DEV
            recv_idx = (my + N_DEV - s - 1) % N_DEV
            slot = s % 2
            rdma = pltpu.make_async_remote_copy(
                src_ref=out_ref.at[pl.ds(send_idx * chunk, chunk), :],
                dst_ref=comm_ref.at[slot],
                send_sem=send_sems.at[s],
                recv_sem=recv_sems.at[s],
                device_id=(right,),
                device_id_type=pl.DeviceIdType.MESH,
            )
            rdma.start()
            rdma.wait()
            acc = out_ref[pl.ds(recv_idx * chunk, chunk), :] + comm_ref[slot]
            out_ref[pl.ds(recv_idx * chunk, chunk), :] = acc

        for s in range(N_DEV - 1):
            send_idx = (my + N_DEV + 1 - s) % N_DEV
            rdma = pltpu.make_async_remote_copy(
                src_ref=out_ref.at[pl.ds(send_idx * chunk, chunk), :],
                dst_ref=out_ref.at[pl.ds(send_idx * chunk, chunk), :],
                send_sem=send_sems.at[N_DEV - 1 + s],
                recv_sem=recv_sems.at[N_DEV - 1 + s],
                device_id=(right,),
                device_id_type=pl.DeviceIdType.MESH,
            )
            rdma.start()
            rdma.wait()

    return pl.pallas_call(
        body,
        out_shape=jax.ShapeDtypeStruct((t, d), x.dtype),
        in_specs=[pl.BlockSpec(memory_space=pltpu.VMEM)],
        out_specs=pl.BlockSpec(memory_space=pltpu.VMEM),
        scratch_shapes=[
            pltpu.VMEM((2, chunk, d), x.dtype),
            pltpu.SemaphoreType.DMA((n_hops,)),
            pltpu.SemaphoreType.DMA((n_hops,)),
        ],
        compiler_params=pltpu.CompilerParams(collective_id=0),
    )(x)


def kernel(ids, E):
    v_shard = E.shape[0]
    my = lax.axis_index("i")
    local = ids - my * v_shard
    in_range = (local >= 0) & (local < v_shard)
    safe = jnp.where(in_range, local, 0)
    partial = jnp.where(in_range[:, None], E[safe], jnp.float32(0))
    return _ring_allreduce(partial)


# baseline (device time: 3377461 ns/iter reference)
import jax
import jax.numpy as jnp
from jax import lax
from jax.experimental import pallas as pl
from jax.experimental.pallas import tpu as pltpu

N_DEV = 4


def _ring_allreduce(x):
    t, d = x.shape
    chunk = t // N_DEV
    n_hops = 2 * (N_DEV - 1)

    def body(x_hbm, out_ref, comm_ref, send_sems, recv_sems, copy_sem):
        my = lax.axis_index("i")
        left = (my + N_DEV - 1) % N_DEV
        right = (my + 1) % N_DEV

        barrier_sem = pltpu.get_barrier_semaphore()
        for nbr in (left, right):
            pl.semaphore_signal(
                barrier_sem, inc=1,
                device_id=(nbr,), device_id_type=pl.DeviceIdType.MESH,
            )
        pl.semaphore_wait(barrier_sem, 2)

        cp = pltpu.make_async_copy(x_hbm, out_ref, copy_sem)
        cp.start()
        cp.wait()

        for s in range(N_DEV - 1):
            send_idx = (my + N_DEV - s) % N_DEV
            recv_idx = (my + N_DEV - s - 1) % N_DEV
            slot = s % 2
            rdma = pltpu.make_async_remote_copy(
                src_ref=out_ref.at[pl.ds(send_idx * chunk, chunk), :],
                dst_ref=comm_ref.at[slot],
                send_sem=send_sems.at[s],
                recv_sem=recv_sems.at[s],
                device_id=(right,),
                device_id_type=pl.DeviceIdType.MESH,
            )
            rdma.start()
            rdma.wait()
            acc = out_ref[pl.ds(recv_idx * chunk, chunk), :] + comm_ref[slot]
            out_ref[pl.ds(recv_idx * chunk, chunk), :] = acc

        for s in range(N_DEV - 1):
            send_idx = (my + N_DEV + 1 - s) % N_DEV
            rdma = pltpu.make_async_remote_copy(
                src_ref=out_ref.at[pl.ds(send_idx * chunk, chunk), :],
                dst_ref=out_ref.at[pl.ds(send_idx * chunk, chunk), :],
                send_sem=send_sems.at[N_DEV - 1 + s],
                recv_sem=recv_sems.at[N_DEV - 1 + s],
                device_id=(right,),
                device_id_type=pl.DeviceIdType.MESH,
            )
            rdma.start()
            rdma.wait()

    return pl.pallas_call(
        body,
        out_shape=jax.ShapeDtypeStruct((t, d), x.dtype),
        in_specs=[pl.BlockSpec(memory_space=pl.ANY)],
        out_specs=pl.BlockSpec(memory_space=pltpu.VMEM),
        scratch_shapes=[
            pltpu.VMEM((2, chunk, d), x.dtype),
            pltpu.SemaphoreType.DMA((n_hops,)),
            pltpu.SemaphoreType.DMA((n_hops,)),
            pltpu.SemaphoreType.DMA,
        ],
        compiler_params=pltpu.CompilerParams(
            collective_id=0, vmem_limit_bytes=56 * 1024 * 1024
        ),
    )(x)


def kernel(ids, E):
    v_shard = E.shape[0]
    my = lax.axis_index("i")
    local = ids - my * v_shard
    in_range = (local >= 0) & (local < v_shard)
    safe = jnp.where(in_range, local, 0)
    partial = jnp.where(in_range[:, None], E[safe], jnp.float32(0))
    return _ring_allreduce(partial)


# device time: 727408 ns/iter; 4.6431x vs baseline; 4.6431x over previous
import jax
import jax.numpy as jnp
from jax import lax
from jax.experimental import pallas as pl
from jax.experimental.pallas import tpu as pltpu

N_DEV = 4


def _fused_lookup_allreduce(safe_ids, mask, E):
    t = safe_ids.shape[0]
    d = E.shape[1]
    chunk = t // N_DEV
    n_hops = 2 * (N_DEV - 1)

    def body(ids_ref, mask_ref, e_hbm, out_ref,
             comm_ref, send_sems, recv_sems, gather_sem):
        my = lax.axis_index("i")
        left = (my + N_DEV - 1) % N_DEV
        right = (my + 1) % N_DEV

        barrier_sem = pltpu.get_barrier_semaphore()
        for nbr in (left, right):
            pl.semaphore_signal(
                barrier_sem, inc=1,
                device_id=(nbr,), device_id_type=pl.DeviceIdType.MESH,
            )
        pl.semaphore_wait(barrier_sem, 2)

        def issue(i, _):
            row = ids_ref[i]
            pltpu.make_async_copy(
                e_hbm.at[row], out_ref.at[i], gather_sem
            ).start()
            return _

        lax.fori_loop(0, t, issue, None)

        def drain(i, _):
            pltpu.make_async_copy(
                e_hbm.at[0], out_ref.at[0], gather_sem
            ).wait()
            return _

        lax.fori_loop(0, t, drain, None)

        out_ref[...] = out_ref[...] * mask_ref[...]

        for s in range(N_DEV - 1):
            send_idx = (my + N_DEV - s) % N_DEV
            recv_idx = (my + N_DEV - s - 1) % N_DEV
            slot = s % 2
            rdma = pltpu.make_async_remote_copy(
                src_ref=out_ref.at[pl.ds(send_idx * chunk, chunk), :],
                dst_ref=comm_ref.at[slot],
                send_sem=send_sems.at[s],
                recv_sem=recv_sems.at[s],
                device_id=(right,),
                device_id_type=pl.DeviceIdType.MESH,
            )
            rdma.start()
            rdma.wait()
            acc = out_ref[pl.ds(recv_idx * chunk, chunk), :] + comm_ref[slot]
            out_ref[pl.ds(recv_idx * chunk, chunk), :] = acc

        for s in range(N_DEV - 1):
            send_idx = (my + N_DEV + 1 - s) % N_DEV
            rdma = pltpu.make_async_remote_copy(
                src_ref=out_ref.at[pl.ds(send_idx * chunk, chunk), :],
                dst_ref=out_ref.at[pl.ds(send_idx * chunk, chunk), :],
                send_sem=send_sems.at[N_DEV - 1 + s],
                recv_sem=recv_sems.at[N_DEV - 1 + s],
                device_id=(right,),
                device_id_type=pl.DeviceIdType.MESH,
            )
            rdma.start()
            rdma.wait()

    return pl.pallas_call(
        body,
        out_shape=jax.ShapeDtypeStruct((t, d), E.dtype),
        in_specs=[
            pl.BlockSpec(memory_space=pltpu.SMEM),
            pl.BlockSpec(memory_space=pltpu.VMEM),
            pl.BlockSpec(memory_space=pl.ANY),
        ],
        out_specs=pl.BlockSpec(memory_space=pltpu.VMEM),
        scratch_shapes=[
            pltpu.VMEM((2, chunk, d), E.dtype),
            pltpu.SemaphoreType.DMA((n_hops,)),
            pltpu.SemaphoreType.DMA((n_hops,)),
            pltpu.SemaphoreType.DMA,
        ],
        compiler_params=pltpu.CompilerParams(
            collective_id=0, vmem_limit_bytes=56 * 1024 * 1024
        ),
    )(safe_ids, mask, E)


def kernel(ids, E):
    v_shard = E.shape[0]
    my = lax.axis_index("i")
    local = ids - my * v_shard
    in_range = (local >= 0) & (local < v_shard)
    safe = jnp.where(in_range, local, 0).astype(jnp.int32)
    mask = in_range.astype(jnp.float32)[:, None]
    return _fused_lookup_allreduce(safe, mask, E)


# device time: 439409 ns/iter; 7.6864x vs baseline; 1.6554x over previous
import jax
import jax.numpy as jnp
from jax import lax
from jax.experimental import pallas as pl
from jax.experimental.pallas import tpu as pltpu

N_DEV = 4


def _fused_lookup_allreduce(safe_ids, flags, count, E):
    t = safe_ids.shape[0]
    d = E.shape[1]
    chunk = t // N_DEV
    dh = d // 2
    n_hops = 2 * (N_DEV - 1)

    def body(ids_ref, flags_ref, count_ref, e_hbm, out_ref,
             comm_r, comm_l, ss_r, rs_r, ss_l, rs_l, gather_sem):
        my = lax.axis_index("i")
        left = (my + N_DEV - 1) % N_DEV
        right = (my + 1) % N_DEV

        barrier_sem = pltpu.get_barrier_semaphore()
        for nbr in (left, right):
            pl.semaphore_signal(
                barrier_sem, inc=1,
                device_id=(nbr,), device_id_type=pl.DeviceIdType.MESH,
            )
        pl.semaphore_wait(barrier_sem, 2)

        out_ref[...] = jnp.zeros((t, d), E.dtype)

        def issue(i, carry):
            @pl.when(flags_ref[i] == 1)
            def _():
                pltpu.make_async_copy(
                    e_hbm.at[ids_ref[i]], out_ref.at[i], gather_sem
                ).start()
            return carry

        lax.fori_loop(0, t, issue, 0)

        def drain(i, carry):
            pltpu.make_async_copy(
                e_hbm.at[0], out_ref.at[0], gather_sem
            ).wait()
            return carry

        lax.fori_loop(0, count_ref[0], drain, 0)

        for s in range(N_DEV - 1):
            si_r = (my + N_DEV - s) % N_DEV
            ri_r = (my + N_DEV - s - 1) % N_DEV
            si_l = (my + s) % N_DEV
            ri_l = (my + s + 1) % N_DEV
            slot = s % 2
            rd_r = pltpu.make_async_remote_copy(
                src_ref=out_ref.at[pl.ds(si_r * chunk, chunk), pl.ds(0, dh)],
                dst_ref=comm_r.at[slot],
                send_sem=ss_r.at[s],
                recv_sem=rs_r.at[s],
                device_id=(right,),
                device_id_type=pl.DeviceIdType.MESH,
            )
            rd_l = pltpu.make_async_remote_copy(
                src_ref=out_ref.at[pl.ds(si_l * chunk, chunk), pl.ds(dh, dh)],
                dst_ref=comm_l.at[slot],
                send_sem=ss_l.at[s],
                recv_sem=rs_l.at[s],
                device_id=(left,),
                device_id_type=pl.DeviceIdType.MESH,
            )
            rd_r.start()
            rd_l.start()
            rd_r.wait()
            rd_l.wait()
            acc_r = out_ref[pl.ds(ri_r * chunk, chunk), pl.ds(0, dh)]
            out_ref[pl.ds(ri_r * chunk, chunk), pl.ds(0, dh)] = (
                acc_r + comm_r[slot]
            )
            acc_l = out_ref[pl.ds(ri_l * chunk, chunk), pl.ds(dh, dh)]
            out_ref[pl.ds(ri_l * chunk, chunk), pl.ds(dh, dh)] = (
                acc_l + comm_l[slot]
            )

        for s in range(N_DEV - 1):
            si_r = (my + N_DEV + 1 - s) % N_DEV
            si_l = (my + N_DEV - 1 + s) % N_DEV
            rd_r = pltpu.make_async_remote_copy(
                src_ref=out_ref.at[pl.ds(si_r * chunk, chunk), pl.ds(0, dh)],
                dst_ref=out_ref.at[pl.ds(si_r * chunk, chunk), pl.ds(0, dh)],
                send_sem=ss_r.at[N_DEV - 1 + s],
                recv_sem=rs_r.at[N_DEV - 1 + s],
                device_id=(right,),
                device_id_type=pl.DeviceIdType.MESH,
            )
            rd_l = pltpu.make_async_remote_copy(
                src_ref=out_ref.at[pl.ds(si_l * chunk, chunk), pl.ds(dh, dh)],
                dst_ref=out_ref.at[pl.ds(si_l * chunk, chunk), pl.ds(dh, dh)],
                send_sem=ss_l.at[N_DEV - 1 + s],
                recv_sem=rs_l.at[N_DEV - 1 + s],
                device_id=(left,),
                device_id_type=pl.DeviceIdType.MESH,
            )
            rd_r.start()
            rd_l.start()
            rd_r.wait()
            rd_l.wait()

    return pl.pallas_call(
        body,
        out_shape=jax.ShapeDtypeStruct((t, d), E.dtype),
        in_specs=[
            pl.BlockSpec(memory_space=pltpu.SMEM),
            pl.BlockSpec(memory_space=pltpu.SMEM),
            pl.BlockSpec(memory_space=pltpu.SMEM),
            pl.BlockSpec(memory_space=pl.ANY),
        ],
        out_specs=pl.BlockSpec(memory_space=pltpu.VMEM),
        scratch_shapes=[
            pltpu.VMEM((2, chunk, dh), E.dtype),
            pltpu.VMEM((2, chunk, dh), E.dtype),
            pltpu.SemaphoreType.DMA((n_hops,)),
            pltpu.SemaphoreType.DMA((n_hops,)),
            pltpu.SemaphoreType.DMA((n_hops,)),
            pltpu.SemaphoreType.DMA((n_hops,)),
            pltpu.SemaphoreType.DMA,
        ],
        compiler_params=pltpu.CompilerParams(
            collective_id=0, vmem_limit_bytes=56 * 1024 * 1024
        ),
    )(safe_ids, flags, count, E)


def kernel(ids, E):
    v_shard = E.shape[0]
    my = lax.axis_index("i")
    local = ids - my * v_shard
    in_range = (local >= 0) & (local < v_shard)
    safe = jnp.where(in_range, local, 0).astype(jnp.int32)
    flags = in_range.astype(jnp.int32)
    count = jnp.sum(flags, dtype=jnp.int32).reshape(1)
    return _fused_lookup_allreduce(safe, flags, count, E)


# device time: 342334 ns/iter; 9.8660x vs baseline; 1.2836x over previous
import jax
import jax.numpy as jnp
from jax import lax
from jax.experimental import pallas as pl
from jax.experimental.pallas import tpu as pltpu

N_DEV = 4


def _fused_lookup_allreduce(src_rows, dst_rows, counts, E, t):
    d = E.shape[1]
    chunk = t // N_DEV
    dh = d // 2
    n_hops = 2 * (N_DEV - 1)

    def body(src_ref, dst_ref, counts_ref, e_hbm, out_ref,
             comm_r, comm_l, ss_r, rs_r, ss_l, rs_l, gsem_ab, gsem_c):
        my = lax.axis_index("i")
        left = (my + N_DEV - 1) % N_DEV
        right = (my + 1) % N_DEV
        n_a = counts_ref[0]
        n_b = counts_ref[1]
        n_c = counts_ref[2]

        barrier_sem = pltpu.get_barrier_semaphore()
        for nbr in (left, right):
            pl.semaphore_signal(
                barrier_sem, inc=1,
                device_id=(nbr,), device_id_type=pl.DeviceIdType.MESH,
            )
        pl.semaphore_wait(barrier_sem, 2)

        out_ref[...] = jnp.zeros((t, d), E.dtype)

        def make_issue(sem):
            def issue(i, carry):
                pltpu.make_async_copy(
                    e_hbm.at[src_ref[i]], out_ref.at[dst_ref[i]], sem
                ).start()
                return carry
            return issue

        def drain(sem, n):
            def one(i, carry):
                pltpu.make_async_copy(
                    e_hbm.at[0], out_ref.at[0], sem
                ).wait()
                return carry
            lax.fori_loop(0, n, one, 0)

        def hop_rdmas(s, si_r, si_l, dst_r, dst_l):
            rd_r = pltpu.make_async_remote_copy(
                src_ref=out_ref.at[pl.ds(si_r * chunk, chunk), pl.ds(0, dh)],
                dst_ref=dst_r,
                send_sem=ss_r.at[s],
                recv_sem=rs_r.at[s],
                device_id=(right,),
                device_id_type=pl.DeviceIdType.MESH,
            )
            rd_l = pltpu.make_async_remote_copy(
                src_ref=out_ref.at[pl.ds(si_l * chunk, chunk), pl.ds(dh, dh)],
                dst_ref=dst_l,
                send_sem=ss_l.at[s],
                recv_sem=rs_l.at[s],
                device_id=(left,),
                device_id_type=pl.DeviceIdType.MESH,
            )
            rd_r.start()
            rd_l.start()
            return rd_r, rd_l

        def accumulate(s, slot):
            ri_r = (my + N_DEV - s - 1) % N_DEV
            ri_l = (my + s + 1) % N_DEV
            acc_r = out_ref[pl.ds(ri_r * chunk, chunk), pl.ds(0, dh)]
            out_ref[pl.ds(ri_r * chunk, chunk), pl.ds(0, dh)] = (
                acc_r + comm_r[slot]
            )
            acc_l = out_ref[pl.ds(ri_l * chunk, chunk), pl.ds(dh, dh)]
            out_ref[pl.ds(ri_l * chunk, chunk), pl.ds(dh, dh)] = (
                acc_l + comm_l[slot]
            )

        lax.fori_loop(0, n_a, make_issue(gsem_ab), 0)
        drain(gsem_ab, n_a)

        rd_r, rd_l = hop_rdmas(0, my, my, comm_r.at[0], comm_l.at[0])

        lax.fori_loop(n_a, n_a + n_b, make_issue(gsem_ab), 0)
        lax.fori_loop(n_a + n_b, n_a + n_b + n_c, make_issue(gsem_c), 0)
        drain(gsem_ab, n_b)
        rd_r.wait()
        rd_l.wait()
        accumulate(0, 0)

        for s in range(1, N_DEV - 1):
            si_r = (my + N_DEV - s) % N_DEV
            si_l = (my + s) % N_DEV
            slot = s % 2
            rd_r, rd_l = hop_rdmas(
                s, si_r, si_l, comm_r.at[slot], comm_l.at[slot]
            )
            if s == 1:
                drain(gsem_c, n_c)
            rd_r.wait()
            rd_l.wait()
            accumulate(s, slot)

        for s in range(N_DEV - 1):
            si_r = (my + N_DEV + 1 - s) % N_DEV
            si_l = (my + N_DEV - 1 + s) % N_DEV
            rd_r, rd_l = hop_rdmas(
                N_DEV - 1 + s,
                si_r, si_l,
                out_ref.at[pl.ds(si_r * chunk, chunk), pl.ds(0, dh)],
                out_ref.at[pl.ds(si_l * chunk, chunk), pl.ds(dh, dh)],
            )
            rd_r.wait()
            rd_l.wait()

    return pl.pallas_call(
        body,
        out_shape=jax.ShapeDtypeStruct((t, d), E.dtype),
        in_specs=[
            pl.BlockSpec(memory_space=pltpu.SMEM),
            pl.BlockSpec(memory_space=pltpu.SMEM),
            pl.BlockSpec(memory_space=pltpu.SMEM),
            pl.BlockSpec(memory_space=pl.ANY),
        ],
        out_specs=pl.BlockSpec(memory_space=pltpu.VMEM),
        scratch_shapes=[
            pltpu.VMEM((2, chunk, dh), E.dtype),
            pltpu.VMEM((2, chunk, dh), E.dtype),
            pltpu.SemaphoreType.DMA((n_hops,)),
            pltpu.SemaphoreType.DMA((n_hops,)),
            pltpu.SemaphoreType.DMA((n_hops,)),
            pltpu.SemaphoreType.DMA((n_hops,)),
            pltpu.SemaphoreType.DMA,
            pltpu.SemaphoreType.DMA,
        ],
        compiler_params=pltpu.CompilerParams(
            collective_id=0, vmem_limit_bytes=56 * 1024 * 1024
        ),
    )(src_rows, dst_rows, counts, E)


def kernel(ids, E):
    t = ids.shape[0]
    v_shard = E.shape[0]
    chunk = t // N_DEV
    my = lax.axis_index("i")
    local = ids - my * v_shard
    in_range = (local >= 0) & (local < v_shard)
    safe = jnp.where(in_range, local, 0).astype(jnp.int32)

    pos = jnp.arange(t, dtype=jnp.int32)
    rel = (pos // chunk - my) % N_DEV
    prio = jnp.where(rel == 0, 0, jnp.where(rel == 2, 2, 1))
    key = jnp.where(in_range, prio, 3)
    order = jnp.argsort(key, stable=True).astype(jnp.int32)
    src_rows = safe[order]
    counts = jnp.stack(
        [jnp.sum(key == k, dtype=jnp.int32) for k in (0, 1, 2)]
    )
    return _fused_lookup_allreduce(src_rows, order, counts, E, t)
